# baseline (speedup 1.0000x reference)
import jax
import jax.numpy as jnp
from jax.experimental import pallas as pl
from jax.experimental.pallas import tpu as pltpu


def _pool_kernel(tok_ref, leni_ref, lenf_ref, emb_hbm, out_ref, emb_vmem, sem):
    c = pl.program_id(0)
    j = pl.program_id(1)
    nj = pl.num_programs(1)
    tb, _, D = out_ref.shape
    seq_len = tok_ref.shape[1]
    chunk = min(16, seq_len)
    n_chunks = seq_len // chunk

    @pl.when(j == 0)
    def _():
        cp = pltpu.make_async_copy(emb_hbm, emb_vmem.at[:, 0], sem)
        cp.start()
        cp.wait()

    base = (c * nj + j) * tb

    def row_body(r, carry):
        b = base + r

        def chunk_sum(a, ci):
            for l in range(ci * chunk, (ci + 1) * chunk):
                a = a + emb_vmem[tok_ref[b, l]]
            return a

        acc = chunk_sum(jnp.zeros((1, D), jnp.float32), 0)
        mylen = leni_ref[b]
        for ci in range(1, n_chunks):
            acc = jax.lax.cond(
                mylen > ci * chunk,
                lambda a, ci=ci: chunk_sum(a, ci),
                lambda a: a,
                acc,
            )
        out_ref[r] = acc / lenf_ref[b]
        return carry

    jax.lax.fori_loop(0, tb, row_body, 0)


def kernel(tok_batch, tok_lens, emb_table):
    B, L = tok_batch.shape
    V, D = emb_table.shape

    n_cores = 2
    tb = 128
    if B % (n_cores * tb) != 0:
        tb = B // n_cores
    tiles_per_core = B // (n_cores * tb)

    tok_i32 = tok_batch.astype(jnp.int32)
    lens_i32 = tok_lens.astype(jnp.int32)
    lens_f32 = tok_lens.astype(jnp.float32)
    emb2 = emb_table.astype(jnp.float32)

    grid_spec = pltpu.PrefetchScalarGridSpec(
        num_scalar_prefetch=3,
        grid=(n_cores, tiles_per_core),
        in_specs=[pl.BlockSpec(memory_space=pl.ANY)],
        out_specs=pl.BlockSpec(
            (tb, 1, D), lambda c, j, tok, li, lf: (c * tiles_per_core + j, 0, 0)
        ),
        scratch_shapes=[
            pltpu.VMEM((V, 1, D), jnp.float32),
            pltpu.SemaphoreType.DMA,
        ],
    )

    out = pl.pallas_call(
        _pool_kernel,
        out_shape=jax.ShapeDtypeStruct((B, 1, D), jnp.float32),
        grid_spec=grid_spec,
        compiler_params=pltpu.CompilerParams(
            dimension_semantics=("parallel", "arbitrary"),
            vmem_limit_bytes=44 << 20,
        ),
    )(tok_i32, lens_i32, lens_f32, emb2)
    return out.reshape(B, D)


# speedup vs baseline: 1.4384x; 1.4384x over previous
import jax
import jax.numpy as jnp
from jax.experimental import pallas as pl
from jax.experimental.pallas import tpu as pltpu


def _pool_kernel(tok_ref, leni_ref, lenf_ref, emb_hbm, out_ref, emb_vmem, sem):
    c = pl.program_id(0)
    j = pl.program_id(1)
    nj = pl.num_programs(1)
    tb, _, D = out_ref.shape
    seq_len = tok_ref.shape[1]
    chunk = min(16, seq_len)
    n_chunks = seq_len // chunk

    @pl.when(j == 0)
    def _():
        cp = pltpu.make_async_copy(emb_hbm, emb_vmem.at[:, 0], sem)
        cp.start()
        cp.wait()

    base = (c * nj + j) * tb

    G = 4

    def group_body(g, carry):
        b0 = base + g * G
        rows = [b0 + i for i in range(G)]
        accs = [emb_vmem[tok_ref[rows[i], 0]] for i in range(G)]
        for l in range(1, seq_len):
            for i in range(G):
                accs[i] = accs[i] + emb_vmem[tok_ref[rows[i], l]]
        for i in range(G):
            out_ref[g * G + i] = accs[i] / lenf_ref[rows[i]]
        return carry

    jax.lax.fori_loop(0, tb // G, group_body, 0)


def kernel(tok_batch, tok_lens, emb_table):
    B, L = tok_batch.shape
    V, D = emb_table.shape

    n_cores = 2
    tb = 128
    if B % (n_cores * tb) != 0:
        tb = B // n_cores
    tiles_per_core = B // (n_cores * tb)

    tok_i32 = tok_batch.astype(jnp.int32)
    lens_i32 = tok_lens.astype(jnp.int32)
    lens_f32 = tok_lens.astype(jnp.float32)
    emb2 = emb_table.astype(jnp.float32)

    grid_spec = pltpu.PrefetchScalarGridSpec(
        num_scalar_prefetch=3,
        grid=(n_cores, tiles_per_core),
        in_specs=[pl.BlockSpec(memory_space=pl.ANY)],
        out_specs=pl.BlockSpec(
            (tb, 1, D), lambda c, j, tok, li, lf: (c * tiles_per_core + j, 0, 0)
        ),
        scratch_shapes=[
            pltpu.VMEM((V, 1, D), jnp.float32),
            pltpu.SemaphoreType.DMA,
        ],
    )

    out = pl.pallas_call(
        _pool_kernel,
        out_shape=jax.ShapeDtypeStruct((B, 1, D), jnp.float32),
        grid_spec=grid_spec,
        compiler_params=pltpu.CompilerParams(
            dimension_semantics=("parallel", "arbitrary"),
            vmem_limit_bytes=44 << 20,
        ),
    )(tok_i32, lens_i32, lens_f32, emb2)
    return out.reshape(B, D)


# speedup vs baseline: 1.4758x; 1.0260x over previous
import jax
import jax.numpy as jnp
from jax.experimental import pallas as pl
from jax.experimental.pallas import tpu as pltpu


def _pool_kernel(tok_ref, leni_ref, lenf_ref, emb_hbm, out_ref, emb_vmem, sem):
    c = pl.program_id(0)
    j = pl.program_id(1)
    nj = pl.num_programs(1)
    tb, _, D = out_ref.shape
    seq_len = tok_ref.shape[1]
    chunk = min(16, seq_len)
    n_chunks = seq_len // chunk

    @pl.when(j == 0)
    def _():
        cp = pltpu.make_async_copy(emb_hbm, emb_vmem.at[:, 0], sem)
        cp.start()
        cp.wait()

    base = (c * nj + j) * tb

    G = 8

    def group_body(g, carry):
        b0 = base + g * G
        rows = [b0 + i for i in range(G)]
        accs = [emb_vmem[tok_ref[rows[i], 0]] for i in range(G)]
        for l in range(1, seq_len):
            for i in range(G):
                accs[i] = accs[i] + emb_vmem[tok_ref[rows[i], l]]
        for i in range(G):
            out_ref[g * G + i] = accs[i] / lenf_ref[rows[i]]
        return carry

    jax.lax.fori_loop(0, tb // G, group_body, 0)


def kernel(tok_batch, tok_lens, emb_table):
    B, L = tok_batch.shape
    V, D = emb_table.shape

    n_cores = 2
    tb = 128
    if B % (n_cores * tb) != 0:
        tb = B // n_cores
    tiles_per_core = B // (n_cores * tb)

    tok_i32 = tok_batch.astype(jnp.int32)
    lens_i32 = tok_lens.astype(jnp.int32)
    lens_f32 = tok_lens.astype(jnp.float32)
    emb2 = emb_table.astype(jnp.float32)

    grid_spec = pltpu.PrefetchScalarGridSpec(
        num_scalar_prefetch=3,
        grid=(n_cores, tiles_per_core),
        in_specs=[pl.BlockSpec(memory_space=pl.ANY)],
        out_specs=pl.BlockSpec(
            (tb, 1, D), lambda c, j, tok, li, lf: (c * tiles_per_core + j, 0, 0)
        ),
        scratch_shapes=[
            pltpu.VMEM((V, 1, D), jnp.float32),
            pltpu.SemaphoreType.DMA,
        ],
    )

    out = pl.pallas_call(
        _pool_kernel,
        out_shape=jax.ShapeDtypeStruct((B, 1, D), jnp.float32),
        grid_spec=grid_spec,
        compiler_params=pltpu.CompilerParams(
            dimension_semantics=("parallel", "arbitrary"),
            vmem_limit_bytes=44 << 20,
        ),
    )(tok_i32, lens_i32, lens_f32, emb2)
    return out.reshape(B, D)


# speedup vs baseline: 1.4822x; 1.0043x over previous
import jax
import jax.numpy as jnp
from jax.experimental import pallas as pl
from jax.experimental.pallas import tpu as pltpu


def _pool_kernel(tok_ref, leni_ref, lenf_ref, emb_hbm, out_ref, emb_vmem, sem):
    c = pl.program_id(0)
    j = pl.program_id(1)
    nj = pl.num_programs(1)
    tb, _, D = out_ref.shape
    seq_len = tok_ref.shape[1]
    chunk = min(16, seq_len)
    n_chunks = seq_len // chunk

    @pl.when(j == 0)
    def _():
        cp = pltpu.make_async_copy(emb_hbm, emb_vmem.at[:, 0], sem)
        cp.start()
        cp.wait()

    base = (c * nj + j) * tb

    G = 16

    def group_body(g, carry):
        b0 = base + g * G
        rows = [b0 + i for i in range(G)]
        accs = [emb_vmem[tok_ref[rows[i], 0]] for i in range(G)]
        for l in range(1, seq_len):
            for i in range(G):
                accs[i] = accs[i] + emb_vmem[tok_ref[rows[i], l]]
        for i in range(G):
            out_ref[g * G + i] = accs[i] / lenf_ref[rows[i]]
        return carry

    jax.lax.fori_loop(0, tb // G, group_body, 0)


def kernel(tok_batch, tok_lens, emb_table):
    B, L = tok_batch.shape
    V, D = emb_table.shape

    n_cores = 2
    tb = 128
    if B % (n_cores * tb) != 0:
        tb = B // n_cores
    tiles_per_core = B // (n_cores * tb)

    tok_i32 = tok_batch.astype(jnp.int32)
    lens_i32 = tok_lens.astype(jnp.int32)
    lens_f32 = tok_lens.astype(jnp.float32)
    emb2 = emb_table.astype(jnp.float32)

    grid_spec = pltpu.PrefetchScalarGridSpec(
        num_scalar_prefetch=3,
        grid=(n_cores, tiles_per_core),
        in_specs=[pl.BlockSpec(memory_space=pl.ANY)],
        out_specs=pl.BlockSpec(
            (tb, 1, D), lambda c, j, tok, li, lf: (c * tiles_per_core + j, 0, 0)
        ),
        scratch_shapes=[
            pltpu.VMEM((V, 1, D), jnp.float32),
            pltpu.SemaphoreType.DMA,
        ],
    )

    out = pl.pallas_call(
        _pool_kernel,
        out_shape=jax.ShapeDtypeStruct((B, 1, D), jnp.float32),
        grid_spec=grid_spec,
        compiler_params=pltpu.CompilerParams(
            dimension_semantics=("parallel", "arbitrary"),
            vmem_limit_bytes=44 << 20,
        ),
    )(tok_i32, lens_i32, lens_f32, emb2)
    return out.reshape(B, D)
